# Initial kernel scaffold; baseline (speedup 1.0000x reference)
import functools

import jax
import jax.numpy as jnp
from jax import lax
from jax.experimental import pallas as pl
from jax.experimental.pallas import tpu as pltpu
from jax.experimental.pallas import tpu_sc as plsc

N = 10000
E = 320000
F = 128
NCLASS = 47

NC = 2
NS = 16
NW = NC * NS

K = 80
EPW = E // NW
CPW = EPW // K
RCH = 80
NRC = N // RCH


def _fill_vmem_2d(ref, nrow, ncol, value):
    def row(i, _):
        def col(j, _):
            ref[i, pl.ds(j * 16, 16)] = jnp.full((16,), value, jnp.float32)
            return 0
        return lax.fori_loop(0, ncol // 16, col, 0)
    lax.fori_loop(0, nrow, row, 0)


def _make_sc_agg(with_cnt):
    out_type = [jax.ShapeDtypeStruct((NC, N, F), jnp.float32)]
    scratch = [
        pltpu.VMEM((K,), jnp.int32),
        pltpu.VMEM((K,), jnp.int32),
        pltpu.VMEM((K, F), jnp.float32),
        pltpu.VMEM_SHARED((N, F), jnp.float32),
        pltpu.SemaphoreType.DMA,
    ]
    if with_cnt:
        out_type.append(jax.ShapeDtypeStruct((NC, N, 16), jnp.float32))
        scratch.append(pltpu.VMEM((K, 16), jnp.float32))
        scratch.append(pltpu.VMEM_SHARED((N, 16), jnp.float32))

    mesh = plsc.VectorSubcoreMesh(core_axis_name="c", subcore_axis_name="s")

    @functools.partial(pl.kernel, mesh=mesh, out_type=tuple(out_type),
                       scratch_types=scratch)
    def agg(h_hbm, ei_hbm, *refs):
        if with_cnt:
            out_hbm, cnt_hbm, sidx, didx, rows, accum, sem, ones_v, cacc = refs
        else:
            out_hbm, sidx, didx, rows, accum, sem = refs

        core = lax.axis_index("c")
        sub = lax.axis_index("s")
        wid = core * NS + sub

        _fill_vmem_2d(rows, K, F, 0.0)
        if with_cnt:
            _fill_vmem_2d(ones_v, K, 16, 0.0)

        def zchunk(t, _):
            c = sub + t * NS
            @pl.when(c < NRC)
            def _():
                pltpu.sync_copy(rows, accum.at[pl.ds(c * RCH, RCH)])
                if with_cnt:
                    pltpu.sync_copy(ones_v, cacc.at[pl.ds(c * RCH, RCH)])
            return 0
        lax.fori_loop(0, (NRC + NS - 1) // NS, zchunk, 0)

        if with_cnt:
            _fill_vmem_2d(ones_v, K, 16, 1.0)

        plsc.subcore_barrier()

        base0 = wid * EPW

        def body(i, _):
            base = base0 + i * K
            pltpu.sync_copy(ei_hbm.at[0, pl.ds(base, K)], sidx)
            pltpu.sync_copy(ei_hbm.at[1, pl.ds(base, K)], didx)
            pltpu.async_copy(h_hbm.at[sidx], rows, sem).wait()
            pltpu.sync_copy(rows, accum.at[didx], add=True)
            if with_cnt:
                pltpu.sync_copy(ones_v, cacc.at[didx], add=True)
            return 0
        lax.fori_loop(0, CPW, body, 0)

        plsc.subcore_barrier()

        def wchunk(t, _):
            c = sub + t * NS
            @pl.when(c < NRC)
            def _():
                pltpu.sync_copy(accum.at[pl.ds(c * RCH, RCH)], rows)
                pltpu.sync_copy(rows, out_hbm.at[core, pl.ds(c * RCH, RCH)])
                if with_cnt:
                    pltpu.sync_copy(cacc.at[pl.ds(c * RCH, RCH)], ones_v)
                    pltpu.sync_copy(ones_v, cnt_hbm.at[core, pl.ds(c * RCH, RCH)])
            return 0
        lax.fori_loop(0, (NRC + NS - 1) // NS, wchunk, 0)

    return agg


_sc_agg_cnt = _make_sc_agg(with_cnt=True)
_sc_agg = _make_sc_agg(with_cnt=False)


BR = 1000


def _pre_body(x_ref, wpre_ref, bpre_ref, wr1_ref, h_ref, r_ref):
    h = jnp.dot(x_ref[...], wpre_ref[...].T,
                preferred_element_type=jnp.float32) + bpre_ref[...]
    h_ref[...] = h
    r_ref[...] = jnp.dot(h, wr1_ref[...].T, preferred_element_type=jnp.float32)


_pre_call = pl.pallas_call(
    _pre_body,
    grid=(N // BR,),
    in_specs=[
        pl.BlockSpec((BR, F), lambda i: (i, 0)),
        pl.BlockSpec((F, F), lambda i: (0, 0)),
        pl.BlockSpec((1, F), lambda i: (0, 0)),
        pl.BlockSpec((F, F), lambda i: (0, 0)),
    ],
    out_specs=[pl.BlockSpec((BR, F), lambda i: (i, 0))] * 2,
    out_shape=[jax.ShapeDtypeStruct((N, F), jnp.float32)] * 2,
)


def _combine1_body(p_ref, c_ref, r_ref, wl_ref, bl_ref, wr2_ref, z_ref, r2_ref):
    s = p_ref[0] + p_ref[1]
    cnt = c_ref[0, :, 0:1] + c_ref[1, :, 0:1]
    agg = s / jnp.maximum(cnt, 1.0)
    z = (jnp.dot(agg, wl_ref[...].T, preferred_element_type=jnp.float32)
         + bl_ref[...] + r_ref[...])
    z = jnp.maximum(z, 0.0)
    z_ref[...] = z
    r2_ref[...] = jnp.dot(z, wr2_ref[...].T, preferred_element_type=jnp.float32)


_combine1_call = pl.pallas_call(
    _combine1_body,
    grid=(N // BR,),
    in_specs=[
        pl.BlockSpec((NC, BR, F), lambda i: (0, i, 0)),
        pl.BlockSpec((NC, BR, 16), lambda i: (0, i, 0)),
        pl.BlockSpec((BR, F), lambda i: (i, 0)),
        pl.BlockSpec((F, F), lambda i: (0, 0)),
        pl.BlockSpec((1, F), lambda i: (0, 0)),
        pl.BlockSpec((F, F), lambda i: (0, 0)),
    ],
    out_specs=[pl.BlockSpec((BR, F), lambda i: (i, 0))] * 2,
    out_shape=[jax.ShapeDtypeStruct((N, F), jnp.float32)] * 2,
)


def _combine2_body(q_ref, c_ref, r_ref, wl_ref, bl_ref, wp_ref, bp_ref, o_ref):
    s = q_ref[0] + q_ref[1]
    cnt = c_ref[0, :, 0:1] + c_ref[1, :, 0:1]
    agg = s / jnp.maximum(cnt, 1.0)
    z = (jnp.dot(agg, wl_ref[...].T, preferred_element_type=jnp.float32)
         + bl_ref[...] + r_ref[...])
    z = jnp.maximum(z, 0.0)
    logits = jnp.dot(z, wp_ref[...].T,
                     preferred_element_type=jnp.float32) + bp_ref[...]
    m = jnp.max(logits, axis=-1, keepdims=True)
    sh = logits - m
    o_ref[...] = sh - jnp.log(jnp.sum(jnp.exp(sh), axis=-1, keepdims=True))


_combine2_call = pl.pallas_call(
    _combine2_body,
    grid=(N // BR,),
    in_specs=[
        pl.BlockSpec((NC, BR, F), lambda i: (0, i, 0)),
        pl.BlockSpec((NC, BR, 16), lambda i: (0, i, 0)),
        pl.BlockSpec((BR, F), lambda i: (i, 0)),
        pl.BlockSpec((F, F), lambda i: (0, 0)),
        pl.BlockSpec((1, F), lambda i: (0, 0)),
        pl.BlockSpec((NCLASS, F), lambda i: (0, 0)),
        pl.BlockSpec((1, NCLASS), lambda i: (0, 0)),
    ],
    out_specs=pl.BlockSpec((BR, NCLASS), lambda i: (i, 0)),
    out_shape=jax.ShapeDtypeStruct((N, NCLASS), jnp.float32),
)


def kernel(x, edge_index, W_pre, b_pre, Wl1, bl1, Wr1, Wl2, bl2, Wr2,
           W_post, b_post):
    h1, r1 = _pre_call(x, W_pre, b_pre.reshape(1, F), Wr1)
    p, c = _sc_agg_cnt(h1, edge_index)
    z1, r2 = _combine1_call(p, c, r1, Wl1, bl1.reshape(1, F), Wr2)
    q = _sc_agg(z1, edge_index)
    out = _combine2_call(q, c, r2, Wl2, bl2.reshape(1, F),
                         W_post, b_post.reshape(1, NCLASS))
    return out


# speedup vs baseline: 4.9270x; 4.9270x over previous
import functools

import jax
import jax.numpy as jnp
from jax import lax
from jax.experimental import pallas as pl
from jax.experimental.pallas import tpu as pltpu
from jax.experimental.pallas import tpu_sc as plsc

N = 10000
E = 320000
F = 128
NCLASS = 47

NC = 2
NS = 16
NW = NC * NS

K = 80
EPW = E // NW
CPW = EPW // K
RCH = 80
NRC = N // RCH


def _fill_vmem_2d(ref, nrow, ncol, value):
    def row(i, _):
        def col(j, _):
            ref[i, pl.ds(j * 16, 16)] = jnp.full((16,), value, jnp.float32)
            return 0
        return lax.fori_loop(0, ncol // 16, col, 0)
    lax.fori_loop(0, nrow, row, 0)


def _fill_vmem_1d(ref, n, value):
    def col(j, _):
        ref[pl.ds(j * 16, 16)] = jnp.full((16,), value, jnp.float32)
        return 0
    lax.fori_loop(0, n // 16, col, 0)


def _make_sc_agg(with_cnt):
    out_type = [jax.ShapeDtypeStruct((NC, N, F), jnp.float32)]
    scratch = [
        pltpu.VMEM((K,), jnp.int32),
        pltpu.VMEM((K,), jnp.int32),
        pltpu.VMEM((K, F), jnp.float32),
        pltpu.VMEM_SHARED((N, F), jnp.float32),
        pltpu.SemaphoreType.DMA,
    ]
    if with_cnt:
        out_type.append(jax.ShapeDtypeStruct((NC * N,), jnp.float32))
        scratch.append(pltpu.VMEM((K,), jnp.float32))
        scratch.append(pltpu.VMEM_SHARED((N,), jnp.float32))

    mesh = plsc.VectorSubcoreMesh(core_axis_name="c", subcore_axis_name="s")

    @functools.partial(pl.kernel, mesh=mesh, out_type=tuple(out_type),
                       scratch_types=scratch)
    def agg(h_hbm, src_hbm, dst_hbm, *refs):
        if with_cnt:
            out_hbm, cnt_hbm, sidx, didx, rows, accum, sem, ones_v, cacc = refs
        else:
            out_hbm, sidx, didx, rows, accum, sem = refs

        core = lax.axis_index("c")
        sub = lax.axis_index("s")
        wid = core * NS + sub

        _fill_vmem_2d(rows, K, F, 0.0)
        if with_cnt:
            _fill_vmem_1d(ones_v, K, 0.0)

        def zchunk(t, _):
            c = sub + t * NS
            @pl.when(c < NRC)
            def _():
                pltpu.sync_copy(rows, accum.at[pl.ds(c * RCH, RCH)])
                if with_cnt:
                    pltpu.sync_copy(ones_v, cacc.at[pl.ds(c * RCH, RCH)])
            return 0
        lax.fori_loop(0, (NRC + NS - 1) // NS, zchunk, 0)

        if with_cnt:
            _fill_vmem_1d(ones_v, K, 1.0)

        plsc.subcore_barrier()

        base0 = wid * EPW

        def body(i, _):
            base = base0 + i * K
            pltpu.sync_copy(src_hbm.at[pl.ds(base, K)], sidx)
            pltpu.sync_copy(dst_hbm.at[pl.ds(base, K)], didx)
            pltpu.async_copy(h_hbm.at[sidx], rows, sem).wait()
            pltpu.sync_copy(rows, accum.at[didx], add=True)
            if with_cnt:
                pltpu.sync_copy(ones_v, cacc.at[didx], add=True)
            return 0
        lax.fori_loop(0, CPW, body, 0)

        plsc.subcore_barrier()

        def wchunk(t, _):
            c = sub + t * NS
            @pl.when(c < NRC)
            def _():
                pltpu.sync_copy(accum.at[pl.ds(c * RCH, RCH)], rows)
                pltpu.sync_copy(rows, out_hbm.at[core, pl.ds(c * RCH, RCH)])
                if with_cnt:
                    pltpu.sync_copy(cacc.at[pl.ds(c * RCH, RCH)], ones_v)
                    pltpu.sync_copy(
                        ones_v, cnt_hbm.at[pl.ds(core * N + c * RCH, RCH)])
            return 0
        lax.fori_loop(0, (NRC + NS - 1) // NS, wchunk, 0)

    return agg


_sc_agg_cnt = _make_sc_agg(with_cnt=True)
_sc_agg = _make_sc_agg(with_cnt=False)


BR = 1000


def _pre_body(x_ref, wpre_ref, bpre_ref, wr1_ref, h_ref, r_ref):
    h = jnp.dot(x_ref[...], wpre_ref[...].T,
                preferred_element_type=jnp.float32) + bpre_ref[...]
    h_ref[...] = h
    r_ref[...] = jnp.dot(h, wr1_ref[...].T, preferred_element_type=jnp.float32)


_pre_call = pl.pallas_call(
    _pre_body,
    grid=(N // BR,),
    in_specs=[
        pl.BlockSpec((BR, F), lambda i: (i, 0)),
        pl.BlockSpec((F, F), lambda i: (0, 0)),
        pl.BlockSpec((1, F), lambda i: (0, 0)),
        pl.BlockSpec((F, F), lambda i: (0, 0)),
    ],
    out_specs=[pl.BlockSpec((BR, F), lambda i: (i, 0))] * 2,
    out_shape=[jax.ShapeDtypeStruct((N, F), jnp.float32)] * 2,
)


def _combine1_body(p_ref, c_ref, r_ref, wl_ref, bl_ref, wr2_ref, z_ref, r2_ref):
    s = p_ref[0] + p_ref[1]
    cnt = c_ref[0] + c_ref[1]
    agg = s / jnp.maximum(cnt, 1.0)
    z = (jnp.dot(agg, wl_ref[...].T, preferred_element_type=jnp.float32)
         + bl_ref[...] + r_ref[...])
    z = jnp.maximum(z, 0.0)
    z_ref[...] = z
    r2_ref[...] = jnp.dot(z, wr2_ref[...].T, preferred_element_type=jnp.float32)


_combine1_call = pl.pallas_call(
    _combine1_body,
    grid=(N // BR,),
    in_specs=[
        pl.BlockSpec((NC, BR, F), lambda i: (0, i, 0)),
        pl.BlockSpec((NC, BR, 1), lambda i: (0, i, 0)),
        pl.BlockSpec((BR, F), lambda i: (i, 0)),
        pl.BlockSpec((F, F), lambda i: (0, 0)),
        pl.BlockSpec((1, F), lambda i: (0, 0)),
        pl.BlockSpec((F, F), lambda i: (0, 0)),
    ],
    out_specs=[pl.BlockSpec((BR, F), lambda i: (i, 0))] * 2,
    out_shape=[jax.ShapeDtypeStruct((N, F), jnp.float32)] * 2,
)


def _combine2_body(q_ref, c_ref, r_ref, wl_ref, bl_ref, wp_ref, bp_ref, o_ref):
    s = q_ref[0] + q_ref[1]
    cnt = c_ref[0] + c_ref[1]
    agg = s / jnp.maximum(cnt, 1.0)
    z = (jnp.dot(agg, wl_ref[...].T, preferred_element_type=jnp.float32)
         + bl_ref[...] + r_ref[...])
    z = jnp.maximum(z, 0.0)
    logits = jnp.dot(z, wp_ref[...].T,
                     preferred_element_type=jnp.float32) + bp_ref[...]
    m = jnp.max(logits, axis=-1, keepdims=True)
    sh = logits - m
    o_ref[...] = sh - jnp.log(jnp.sum(jnp.exp(sh), axis=-1, keepdims=True))


_combine2_call = pl.pallas_call(
    _combine2_body,
    grid=(N // BR,),
    in_specs=[
        pl.BlockSpec((NC, BR, F), lambda i: (0, i, 0)),
        pl.BlockSpec((NC, BR, 1), lambda i: (0, i, 0)),
        pl.BlockSpec((BR, F), lambda i: (i, 0)),
        pl.BlockSpec((F, F), lambda i: (0, 0)),
        pl.BlockSpec((1, F), lambda i: (0, 0)),
        pl.BlockSpec((NCLASS, F), lambda i: (0, 0)),
        pl.BlockSpec((1, NCLASS), lambda i: (0, 0)),
    ],
    out_specs=pl.BlockSpec((BR, NCLASS), lambda i: (i, 0)),
    out_shape=jax.ShapeDtypeStruct((N, NCLASS), jnp.float32),
)


def kernel(x, edge_index, W_pre, b_pre, Wl1, bl1, Wr1, Wl2, bl2, Wr2,
           W_post, b_post):
    src = edge_index[0]
    dst = edge_index[1]
    h1, r1 = _pre_call(x, W_pre, b_pre.reshape(1, F), Wr1)
    p, c = _sc_agg_cnt(h1, src, dst)
    c = c.reshape(NC, N, 1)
    z1, r2 = _combine1_call(p, c, r1, Wl1, bl1.reshape(1, F), Wr2)
    q, = _sc_agg(z1, src, dst)
    out = _combine2_call(q, c, r2, Wl2, bl2.reshape(1, F),
                         W_post, b_post.reshape(1, NCLASS))
    return out


# speedup vs baseline: 10.7792x; 2.1878x over previous
import functools

import jax
import jax.numpy as jnp
from jax import lax
from jax.experimental import pallas as pl
from jax.experimental.pallas import tpu as pltpu
from jax.experimental.pallas import tpu_sc as plsc

N = 10000
E = 320000
F = 128
NCLASS = 47

NC = 2
NS = 16
NW = NC * NS

K = 80
EPW = E // NW
CPW = EPW // K
RCH = 80
NRC = N // RCH

NBUF = 2
GRP = (CPW - 1) // NBUF


def _fill_vmem_2d(ref, nrow, ncol, value):
    def row(i, _):
        def col(j, _):
            ref[i, pl.ds(j * 16, 16)] = jnp.full((16,), value, jnp.float32)
            return 0
        return lax.fori_loop(0, ncol // 16, col, 0)
    lax.fori_loop(0, nrow, row, 0)


def _fill_vmem_1d(ref, n, value):
    def col(j, _):
        ref[pl.ds(j * 16, 16)] = jnp.full((16,), value, jnp.float32)
        return 0
    lax.fori_loop(0, n // 16, col, 0)


def _make_sc_agg(with_cnt):
    out_type = [jax.ShapeDtypeStruct((NC, N, F), jnp.float32)]
    scratch = [
        pltpu.VMEM((EPW,), jnp.int32),
        pltpu.VMEM((NBUF, K), jnp.int32),
        pltpu.VMEM((NBUF, K, F), jnp.float32),
        pltpu.VMEM_SHARED((N, F), jnp.float32),
    ] + [pltpu.SemaphoreType.DMA] * (2 * NBUF)
    if with_cnt:
        out_type.append(jax.ShapeDtypeStruct((NC * N,), jnp.float32))
        scratch.append(pltpu.VMEM((K,), jnp.float32))
        scratch.append(pltpu.VMEM_SHARED((N,), jnp.float32))

    mesh = plsc.VectorSubcoreMesh(core_axis_name="c", subcore_axis_name="s")

    @functools.partial(pl.kernel, mesh=mesh, out_type=tuple(out_type),
                       scratch_types=scratch)
    def agg(h_hbm, src_hbm, dst_hbm, *refs):
        if with_cnt:
            (out_hbm, cnt_hbm, sidx, didx, rows, accum,
             *sems, ones_v, cacc) = refs
        else:
            out_hbm, sidx, didx, rows, accum, *sems = refs
        gsem, dsem = sems[:NBUF], sems[NBUF:]

        core = lax.axis_index("c")
        sub = lax.axis_index("s")
        wid = core * NS + sub

        zbuf = rows.at[0]
        _fill_vmem_2d(zbuf, K, F, 0.0)
        if with_cnt:
            _fill_vmem_1d(ones_v, K, 0.0)

        def zchunk(t, _):
            c = sub + t * NS
            @pl.when(c < NRC)
            def _():
                pltpu.sync_copy(zbuf, accum.at[pl.ds(c * RCH, RCH)])
                if with_cnt:
                    pltpu.sync_copy(ones_v, cacc.at[pl.ds(c * RCH, RCH)])
            return 0
        lax.fori_loop(0, (NRC + NS - 1) // NS, zchunk, 0)

        if with_cnt:
            _fill_vmem_1d(ones_v, K, 1.0)

        ebase = wid * EPW
        pltpu.sync_copy(src_hbm.at[pl.ds(ebase, EPW)], sidx)
        for b in range(NBUF):
            pltpu.async_copy(
                h_hbm.at[sidx.at[pl.ds(b * K, K)]], rows.at[b], gsem[b])
            pltpu.async_copy(
                dst_hbm.at[pl.ds(ebase + b * K, K)], didx.at[b], dsem[b])

        plsc.subcore_barrier()

        def chunk(b, i):
            pltpu.make_async_copy(
                h_hbm.at[sidx.at[pl.ds(0, K)]], rows.at[b], gsem[b]).wait()
            pltpu.make_async_copy(
                dst_hbm.at[pl.ds(0, K)], didx.at[b], dsem[b]).wait()
            pltpu.sync_copy(rows.at[b], accum.at[didx.at[b]], add=True)
            if with_cnt:
                pltpu.sync_copy(ones_v, cacc.at[didx.at[b]], add=True)

        def group(g, _):
            for b in range(NBUF):
                i = g * NBUF + b
                chunk(b, i)
                @pl.when(i + NBUF < CPW)
                def _():
                    pltpu.async_copy(
                        h_hbm.at[sidx.at[pl.ds((i + NBUF) * K, K)]],
                        rows.at[b], gsem[b])
                    pltpu.async_copy(
                        dst_hbm.at[pl.ds(ebase + (i + NBUF) * K, K)],
                        didx.at[b], dsem[b])
            return 0
        lax.fori_loop(0, GRP, group, 0)
        chunk((CPW - 1) % NBUF, CPW - 1)

        plsc.subcore_barrier()

        def wchunk(t, _):
            c = sub + t * NS
            @pl.when(c < NRC)
            def _():
                pltpu.sync_copy(accum.at[pl.ds(c * RCH, RCH)], zbuf)
                pltpu.sync_copy(zbuf, out_hbm.at[core, pl.ds(c * RCH, RCH)])
                if with_cnt:
                    pltpu.sync_copy(cacc.at[pl.ds(c * RCH, RCH)], ones_v)
                    pltpu.sync_copy(
                        ones_v, cnt_hbm.at[pl.ds(core * N + c * RCH, RCH)])
            return 0
        lax.fori_loop(0, (NRC + NS - 1) // NS, wchunk, 0)

    return agg


_sc_agg_cnt = _make_sc_agg(with_cnt=True)
_sc_agg = _make_sc_agg(with_cnt=False)


BR = 1000


def _pre_body(x_ref, wpre_ref, bpre_ref, wr1_ref, h_ref, r_ref):
    h = jnp.dot(x_ref[...], wpre_ref[...].T,
                preferred_element_type=jnp.float32) + bpre_ref[...]
    h_ref[...] = h
    r_ref[...] = jnp.dot(h, wr1_ref[...].T, preferred_element_type=jnp.float32)


_pre_call = pl.pallas_call(
    _pre_body,
    grid=(N // BR,),
    in_specs=[
        pl.BlockSpec((BR, F), lambda i: (i, 0)),
        pl.BlockSpec((F, F), lambda i: (0, 0)),
        pl.BlockSpec((1, F), lambda i: (0, 0)),
        pl.BlockSpec((F, F), lambda i: (0, 0)),
    ],
    out_specs=[pl.BlockSpec((BR, F), lambda i: (i, 0))] * 2,
    out_shape=[jax.ShapeDtypeStruct((N, F), jnp.float32)] * 2,
)


def _combine1_body(p_ref, c_ref, r_ref, wl_ref, bl_ref, wr2_ref, z_ref, r2_ref):
    s = p_ref[0] + p_ref[1]
    cnt = c_ref[0] + c_ref[1]
    agg = s / jnp.maximum(cnt, 1.0)
    z = (jnp.dot(agg, wl_ref[...].T, preferred_element_type=jnp.float32)
         + bl_ref[...] + r_ref[...])
    z = jnp.maximum(z, 0.0)
    z_ref[...] = z
    r2_ref[...] = jnp.dot(z, wr2_ref[...].T, preferred_element_type=jnp.float32)


_combine1_call = pl.pallas_call(
    _combine1_body,
    grid=(N // BR,),
    in_specs=[
        pl.BlockSpec((NC, BR, F), lambda i: (0, i, 0)),
        pl.BlockSpec((NC, BR, 1), lambda i: (0, i, 0)),
        pl.BlockSpec((BR, F), lambda i: (i, 0)),
        pl.BlockSpec((F, F), lambda i: (0, 0)),
        pl.BlockSpec((1, F), lambda i: (0, 0)),
        pl.BlockSpec((F, F), lambda i: (0, 0)),
    ],
    out_specs=[pl.BlockSpec((BR, F), lambda i: (i, 0))] * 2,
    out_shape=[jax.ShapeDtypeStruct((N, F), jnp.float32)] * 2,
)


def _combine2_body(q_ref, c_ref, r_ref, wl_ref, bl_ref, wp_ref, bp_ref, o_ref):
    s = q_ref[0] + q_ref[1]
    cnt = c_ref[0] + c_ref[1]
    agg = s / jnp.maximum(cnt, 1.0)
    z = (jnp.dot(agg, wl_ref[...].T, preferred_element_type=jnp.float32)
         + bl_ref[...] + r_ref[...])
    z = jnp.maximum(z, 0.0)
    logits = jnp.dot(z, wp_ref[...].T,
                     preferred_element_type=jnp.float32) + bp_ref[...]
    m = jnp.max(logits, axis=-1, keepdims=True)
    sh = logits - m
    o_ref[...] = sh - jnp.log(jnp.sum(jnp.exp(sh), axis=-1, keepdims=True))


_combine2_call = pl.pallas_call(
    _combine2_body,
    grid=(N // BR,),
    in_specs=[
        pl.BlockSpec((NC, BR, F), lambda i: (0, i, 0)),
        pl.BlockSpec((NC, BR, 1), lambda i: (0, i, 0)),
        pl.BlockSpec((BR, F), lambda i: (i, 0)),
        pl.BlockSpec((F, F), lambda i: (0, 0)),
        pl.BlockSpec((1, F), lambda i: (0, 0)),
        pl.BlockSpec((NCLASS, F), lambda i: (0, 0)),
        pl.BlockSpec((1, NCLASS), lambda i: (0, 0)),
    ],
    out_specs=pl.BlockSpec((BR, NCLASS), lambda i: (i, 0)),
    out_shape=jax.ShapeDtypeStruct((N, NCLASS), jnp.float32),
)


def kernel(x, edge_index, W_pre, b_pre, Wl1, bl1, Wr1, Wl2, bl2, Wr2,
           W_post, b_post):
    src = edge_index[0]
    dst = edge_index[1]
    h1, r1 = _pre_call(x, W_pre, b_pre.reshape(1, F), Wr1)
    p, c = _sc_agg_cnt(h1, src, dst)
    c = c.reshape(NC, N, 1)
    z1, r2 = _combine1_call(p, c, r1, Wl1, bl1.reshape(1, F), Wr2)
    q, = _sc_agg(z1, src, dst)
    out = _combine2_call(q, c, r2, Wl2, bl2.reshape(1, F),
                         W_post, b_post.reshape(1, NCLASS))
    return out


# speedup vs baseline: 12.5949x; 1.1684x over previous
import functools

import jax
import jax.numpy as jnp
from jax import lax
from jax.experimental import pallas as pl
from jax.experimental.pallas import tpu as pltpu
from jax.experimental.pallas import tpu_sc as plsc

N = 10000
E = 320000
F = 128
NCLASS = 47

NC = 2
NS = 16
NW = NC * NS

K = 80
EPW = E // NW
CPW = EPW // K
RCH = 80
NRC = N // RCH

NBUF = 3
GRP = (CPW - 1) // NBUF


def _fill_vmem_2d(ref, nrow, ncol, value):
    def row(i, _):
        def col(j, _):
            ref[i, pl.ds(j * 16, 16)] = jnp.full((16,), value, jnp.float32)
            return 0
        return lax.fori_loop(0, ncol // 16, col, 0)
    lax.fori_loop(0, nrow, row, 0)


def _fill_vmem_1d(ref, n, value):
    def col(j, _):
        ref[pl.ds(j * 16, 16)] = jnp.full((16,), value, jnp.float32)
        return 0
    lax.fori_loop(0, n // 16, col, 0)


def _make_sc_agg(with_cnt):
    out_type = [jax.ShapeDtypeStruct((NC, N, F), jnp.float32)]
    scratch = [
        pltpu.VMEM((EPW,), jnp.int32),
        pltpu.VMEM((NBUF, K), jnp.int32),
        pltpu.VMEM((NBUF, K, F), jnp.float32),
        pltpu.VMEM_SHARED((N, F), jnp.float32),
    ] + [pltpu.SemaphoreType.DMA] * (2 * NBUF)
    if with_cnt:
        out_type.append(jax.ShapeDtypeStruct((NC * N,), jnp.float32))
        scratch.append(pltpu.VMEM((K,), jnp.float32))
        scratch.append(pltpu.VMEM_SHARED((N,), jnp.float32))

    mesh = plsc.VectorSubcoreMesh(core_axis_name="c", subcore_axis_name="s")

    @functools.partial(pl.kernel, mesh=mesh, out_type=tuple(out_type),
                       scratch_types=scratch)
    def agg(h_hbm, src_hbm, dst_hbm, *refs):
        if with_cnt:
            (out_hbm, cnt_hbm, sidx, didx, rows, accum,
             *sems, ones_v, cacc) = refs
        else:
            out_hbm, sidx, didx, rows, accum, *sems = refs
        gsem, dsem = sems[:NBUF], sems[NBUF:]

        core = lax.axis_index("c")
        sub = lax.axis_index("s")
        wid = core * NS + sub

        zbuf = rows.at[0]
        _fill_vmem_2d(zbuf, K, F, 0.0)
        if with_cnt:
            _fill_vmem_1d(ones_v, K, 0.0)

        def zchunk(t, _):
            c = sub + t * NS
            @pl.when(c < NRC)
            def _():
                pltpu.sync_copy(zbuf, accum.at[pl.ds(c * RCH, RCH)])
                if with_cnt:
                    pltpu.sync_copy(ones_v, cacc.at[pl.ds(c * RCH, RCH)])
            return 0
        lax.fori_loop(0, (NRC + NS - 1) // NS, zchunk, 0)

        if with_cnt:
            _fill_vmem_1d(ones_v, K, 1.0)

        ebase = wid * EPW
        pltpu.sync_copy(src_hbm.at[pl.ds(ebase, EPW)], sidx)
        for b in range(NBUF):
            pltpu.async_copy(
                h_hbm.at[sidx.at[pl.ds(b * K, K)]], rows.at[b], gsem[b])
            pltpu.async_copy(
                dst_hbm.at[pl.ds(ebase + b * K, K)], didx.at[b], dsem[b])

        plsc.subcore_barrier()

        def chunk(b, i):
            pltpu.make_async_copy(
                h_hbm.at[sidx.at[pl.ds(0, K)]], rows.at[b], gsem[b]).wait()
            pltpu.make_async_copy(
                dst_hbm.at[pl.ds(0, K)], didx.at[b], dsem[b]).wait()
            pltpu.sync_copy(rows.at[b], accum.at[didx.at[b]], add=True)
            if with_cnt:
                pltpu.sync_copy(ones_v, cacc.at[didx.at[b]], add=True)

        def group(g, _):
            for b in range(NBUF):
                i = g * NBUF + b
                chunk(b, i)
                @pl.when(i + NBUF < CPW)
                def _():
                    pltpu.async_copy(
                        h_hbm.at[sidx.at[pl.ds((i + NBUF) * K, K)]],
                        rows.at[b], gsem[b])
                    pltpu.async_copy(
                        dst_hbm.at[pl.ds(ebase + (i + NBUF) * K, K)],
                        didx.at[b], dsem[b])
            return 0
        lax.fori_loop(0, GRP, group, 0)
        for i in range(GRP * NBUF, CPW):
            chunk(i % NBUF, i)

        plsc.subcore_barrier()

        def wchunk(t, _):
            c = sub + t * NS
            @pl.when(c < NRC)
            def _():
                pltpu.sync_copy(accum.at[pl.ds(c * RCH, RCH)], zbuf)
                pltpu.sync_copy(zbuf, out_hbm.at[core, pl.ds(c * RCH, RCH)])
                if with_cnt:
                    pltpu.sync_copy(cacc.at[pl.ds(c * RCH, RCH)], ones_v)
                    pltpu.sync_copy(
                        ones_v, cnt_hbm.at[pl.ds(core * N + c * RCH, RCH)])
            return 0
        lax.fori_loop(0, (NRC + NS - 1) // NS, wchunk, 0)

    return agg


_sc_agg_cnt = _make_sc_agg(with_cnt=True)
_sc_agg = _make_sc_agg(with_cnt=False)


BR = 1000


def _pre_body(x_ref, wpre_ref, bpre_ref, wr1_ref, h_ref, r_ref):
    h = jnp.dot(x_ref[...], wpre_ref[...].T,
                preferred_element_type=jnp.float32) + bpre_ref[...]
    h_ref[...] = h
    r_ref[...] = jnp.dot(h, wr1_ref[...].T, preferred_element_type=jnp.float32)


_pre_call = pl.pallas_call(
    _pre_body,
    grid=(N // BR,),
    in_specs=[
        pl.BlockSpec((BR, F), lambda i: (i, 0)),
        pl.BlockSpec((F, F), lambda i: (0, 0)),
        pl.BlockSpec((1, F), lambda i: (0, 0)),
        pl.BlockSpec((F, F), lambda i: (0, 0)),
    ],
    out_specs=[pl.BlockSpec((BR, F), lambda i: (i, 0))] * 2,
    out_shape=[jax.ShapeDtypeStruct((N, F), jnp.float32)] * 2,
)


def _combine1_body(p_ref, c_ref, r_ref, wl_ref, bl_ref, wr2_ref, z_ref, r2_ref):
    s = p_ref[0] + p_ref[1]
    cnt = c_ref[0] + c_ref[1]
    agg = s / jnp.maximum(cnt, 1.0)
    z = (jnp.dot(agg, wl_ref[...].T, preferred_element_type=jnp.float32)
         + bl_ref[...] + r_ref[...])
    z = jnp.maximum(z, 0.0)
    z_ref[...] = z
    r2_ref[...] = jnp.dot(z, wr2_ref[...].T, preferred_element_type=jnp.float32)


_combine1_call = pl.pallas_call(
    _combine1_body,
    grid=(N // BR,),
    in_specs=[
        pl.BlockSpec((NC, BR, F), lambda i: (0, i, 0)),
        pl.BlockSpec((NC, BR, 1), lambda i: (0, i, 0)),
        pl.BlockSpec((BR, F), lambda i: (i, 0)),
        pl.BlockSpec((F, F), lambda i: (0, 0)),
        pl.BlockSpec((1, F), lambda i: (0, 0)),
        pl.BlockSpec((F, F), lambda i: (0, 0)),
    ],
    out_specs=[pl.BlockSpec((BR, F), lambda i: (i, 0))] * 2,
    out_shape=[jax.ShapeDtypeStruct((N, F), jnp.float32)] * 2,
)


def _combine2_body(q_ref, c_ref, r_ref, wl_ref, bl_ref, wp_ref, bp_ref, o_ref):
    s = q_ref[0] + q_ref[1]
    cnt = c_ref[0] + c_ref[1]
    agg = s / jnp.maximum(cnt, 1.0)
    z = (jnp.dot(agg, wl_ref[...].T, preferred_element_type=jnp.float32)
         + bl_ref[...] + r_ref[...])
    z = jnp.maximum(z, 0.0)
    logits = jnp.dot(z, wp_ref[...].T,
                     preferred_element_type=jnp.float32) + bp_ref[...]
    m = jnp.max(logits, axis=-1, keepdims=True)
    sh = logits - m
    o_ref[...] = sh - jnp.log(jnp.sum(jnp.exp(sh), axis=-1, keepdims=True))


_combine2_call = pl.pallas_call(
    _combine2_body,
    grid=(N // BR,),
    in_specs=[
        pl.BlockSpec((NC, BR, F), lambda i: (0, i, 0)),
        pl.BlockSpec((NC, BR, 1), lambda i: (0, i, 0)),
        pl.BlockSpec((BR, F), lambda i: (i, 0)),
        pl.BlockSpec((F, F), lambda i: (0, 0)),
        pl.BlockSpec((1, F), lambda i: (0, 0)),
        pl.BlockSpec((NCLASS, F), lambda i: (0, 0)),
        pl.BlockSpec((1, NCLASS), lambda i: (0, 0)),
    ],
    out_specs=pl.BlockSpec((BR, NCLASS), lambda i: (i, 0)),
    out_shape=jax.ShapeDtypeStruct((N, NCLASS), jnp.float32),
)


def kernel(x, edge_index, W_pre, b_pre, Wl1, bl1, Wr1, Wl2, bl2, Wr2,
           W_post, b_post):
    src = edge_index[0]
    dst = edge_index[1]
    h1, r1 = _pre_call(x, W_pre, b_pre.reshape(1, F), Wr1)
    p, c = _sc_agg_cnt(h1, src, dst)
    c = c.reshape(NC, N, 1)
    z1, r2 = _combine1_call(p, c, r1, Wl1, bl1.reshape(1, F), Wr2)
    q, = _sc_agg(z1, src, dst)
    out = _combine2_call(q, c, r2, Wl2, bl2.reshape(1, F),
                         W_post, b_post.reshape(1, NCLASS))
    return out
